# baseline (speedup 1.0000x reference)
import functools

import jax
import jax.numpy as jnp
from jax import lax
from jax.experimental import pallas as pl
from jax.experimental.pallas import tpu as pltpu
from jax.experimental.pallas import tpu_sc as plsc

VOCAB = 1000
D_MODEL = 64
BATCH = 1024
SEQ = 50

B_TOTAL = BATCH * SEQ
NC, NS = 2, 16
NW = NC * NS
B_PER_W = B_TOTAL // NW
CHUNK = 64
NCHUNKS = B_PER_W // CHUNK


def _matmul_body(emb_ref, w_ref, m_ref):
    m_ref[...] = lax.dot_general(
        emb_ref[...], w_ref[...],
        dimension_numbers=(((1,), (1,)), ((), ())),
        preferred_element_type=jnp.float32,
    )


def _compute_logit_table(embed_table, head_w):
    return pl.pallas_call(
        _matmul_body,
        out_shape=jax.ShapeDtypeStruct((VOCAB, VOCAB), jnp.float32),
    )(embed_table, head_w)


def _gather_body(m_hbm, idx_hbm, out_hbm, idx_v, rows_v, gsem):
    wid = lax.axis_index("s") * NC + lax.axis_index("c")
    base = wid * B_PER_W
    pltpu.sync_copy(idx_hbm.at[pl.ds(base, B_PER_W)], idx_v)

    def step(i, _):
        pltpu.async_copy(
            m_hbm.at[idx_v.at[pl.ds(i * CHUNK, CHUNK)]],
            rows_v.at[0],
            gsem,
        ).wait()
        pltpu.sync_copy(rows_v.at[0], out_hbm.at[pl.ds(base + i * CHUNK, CHUNK)])
        return 0

    lax.fori_loop(0, NCHUNKS, step, 0)


_gather_call = functools.partial(
    pl.kernel,
    out_type=jax.ShapeDtypeStruct((B_TOTAL, VOCAB), jnp.float32),
    mesh=plsc.VectorSubcoreMesh(core_axis_name="c", subcore_axis_name="s"),
    compiler_params=pltpu.CompilerParams(use_tc_tiling_on_sc=False),
    scratch_types=[
        pltpu.VMEM((B_PER_W,), jnp.int32),
        pltpu.VMEM((1, CHUNK, VOCAB), jnp.float32),
        pltpu.SemaphoreType.DMA,
    ],
)(_gather_body)


def kernel(input_ids, embed_table, head_w):
    m = _compute_logit_table(embed_table, head_w)
    idx = input_ids.reshape(-1).astype(jnp.int32)
    out = _gather_call(m, idx)
    return out.reshape(input_ids.shape + (VOCAB,))


# speedup vs baseline: 1.0074x; 1.0074x over previous
import functools

import jax
import jax.numpy as jnp
from jax import lax
from jax.experimental import pallas as pl
from jax.experimental.pallas import tpu as pltpu
from jax.experimental.pallas import tpu_sc as plsc

VOCAB = 1000
D_MODEL = 64
BATCH = 1024
SEQ = 50

B_TOTAL = BATCH * SEQ
NC, NS = 2, 16
NW = NC * NS
B_PER_W = B_TOTAL // NW
NBUF = 4
CHUNK = 16
NROUNDS = B_PER_W // (NBUF * CHUNK)


def _matmul_body(emb_ref, w_ref, m_ref):
    m_ref[...] = lax.dot_general(
        emb_ref[...], w_ref[...],
        dimension_numbers=(((1,), (1,)), ((), ())),
        preferred_element_type=jnp.float32,
    )


def _compute_logit_table(embed_table, head_w):
    return pl.pallas_call(
        _matmul_body,
        out_shape=jax.ShapeDtypeStruct((VOCAB, VOCAB), jnp.float32),
    )(embed_table, head_w)


def _gather_body(m_hbm, idx_hbm, out_hbm, idx_v, rows_v,
                 g0, g1, g2, g3, w0, w1, w2, w3):
    gsems = [g0, g1, g2, g3]
    wsems = [w0, w1, w2, w3]
    wid = lax.axis_index("s") * NC + lax.axis_index("c")
    base = wid * B_PER_W
    pltpu.sync_copy(idx_hbm.at[pl.ds(base, B_PER_W)], idx_v)

    def fire_gather(chunk, s):
        pltpu.async_copy(
            m_hbm.at[idx_v.at[pl.ds(chunk * CHUNK, CHUNK)]],
            rows_v.at[s],
            gsems[s],
        )

    def wait_gather(s):
        pltpu.make_async_copy(
            m_hbm.at[pl.ds(0, CHUNK)], rows_v.at[s], gsems[s]
        ).wait()

    for s in range(NBUF):
        fire_gather(s, s)

    def round_body(j, _):
        first = j * NBUF
        writes = []
        for s in range(NBUF):
            wait_gather(s)
            writes.append(
                pltpu.async_copy(
                    rows_v.at[s],
                    out_hbm.at[pl.ds(base + (first + s) * CHUNK, CHUNK)],
                    wsems[s],
                )
            )
        for s in range(NBUF):
            writes[s].wait()

            @pl.when(j < NROUNDS - 1)
            def _():
                fire_gather(first + NBUF + s, s)

        return 0

    lax.fori_loop(0, NROUNDS, round_body, 0)


_gather_call = functools.partial(
    pl.kernel,
    out_type=jax.ShapeDtypeStruct((B_TOTAL, VOCAB), jnp.float32),
    mesh=plsc.VectorSubcoreMesh(core_axis_name="c", subcore_axis_name="s"),
    compiler_params=pltpu.CompilerParams(use_tc_tiling_on_sc=False),
    scratch_types=[
        pltpu.VMEM((B_PER_W,), jnp.int32),
        pltpu.VMEM((NBUF, CHUNK, VOCAB), jnp.float32),
        pltpu.SemaphoreType.DMA,
        pltpu.SemaphoreType.DMA,
        pltpu.SemaphoreType.DMA,
        pltpu.SemaphoreType.DMA,
        pltpu.SemaphoreType.DMA,
        pltpu.SemaphoreType.DMA,
        pltpu.SemaphoreType.DMA,
        pltpu.SemaphoreType.DMA,
    ],
)(_gather_body)


def kernel(input_ids, embed_table, head_w):
    m = _compute_logit_table(embed_table, head_w)
    idx = input_ids.reshape(-1).astype(jnp.int32)
    out = _gather_call(m, idx)
    return out.reshape(input_ids.shape + (VOCAB,))


# speedup vs baseline: 1.7362x; 1.7234x over previous
import functools

import jax
import jax.numpy as jnp
from jax import lax
from jax.experimental import pallas as pl
from jax.experimental.pallas import tpu as pltpu
from jax.experimental.pallas import tpu_sc as plsc

VOCAB = 1000
D_MODEL = 64
D_PAD = 128
BATCH = 1024
SEQ = 50

B_TOTAL = BATCH * SEQ
NC, NS = 2, 16
NW = NC * NS
B_PER_W = B_TOTAL // NW
NBUF = 4
CHUNK = 80
NROUNDS = B_PER_W // (NBUF * CHUNK)

BB = 8
ROWS_PER_BLK = BB * SEQ


def _gather_body(e_hbm, idx_hbm, x_hbm, idx_v, rows_v,
                 g0, g1, g2, g3, w0, w1, w2, w3):
    gsems = [g0, g1, g2, g3]
    wsems = [w0, w1, w2, w3]
    wid = lax.axis_index("s") * NC + lax.axis_index("c")
    base = wid * B_PER_W
    pltpu.sync_copy(idx_hbm.at[pl.ds(base, B_PER_W)], idx_v)

    def fire_gather(chunk, s):
        pltpu.async_copy(
            e_hbm.at[idx_v.at[pl.ds(chunk * CHUNK, CHUNK)]],
            rows_v.at[s],
            gsems[s],
        )

    def wait_gather(s):
        pltpu.make_async_copy(
            e_hbm.at[pl.ds(0, CHUNK)], rows_v.at[s], gsems[s]
        ).wait()

    for s in range(NBUF):
        fire_gather(s, s)

    def round_body(j, _):
        first = j * NBUF
        writes = []
        for s in range(NBUF):
            wait_gather(s)
            writes.append(
                pltpu.async_copy(
                    rows_v.at[s],
                    x_hbm.at[pl.ds(base + (first + s) * CHUNK, CHUNK)],
                    wsems[s],
                )
            )
        for s in range(NBUF):
            writes[s].wait()

            @pl.when(j < NROUNDS - 1)
            def _():
                fire_gather(first + NBUF + s, s)

        return 0

    lax.fori_loop(0, NROUNDS, round_body, 0)


_gather_call = functools.partial(
    pl.kernel,
    out_type=jax.ShapeDtypeStruct((B_TOTAL, D_PAD), jnp.float32),
    mesh=plsc.VectorSubcoreMesh(core_axis_name="c", subcore_axis_name="s"),
    scratch_types=[
        pltpu.VMEM((B_PER_W,), jnp.int32),
        pltpu.VMEM((NBUF, CHUNK, D_PAD), jnp.float32),
        pltpu.SemaphoreType.DMA,
        pltpu.SemaphoreType.DMA,
        pltpu.SemaphoreType.DMA,
        pltpu.SemaphoreType.DMA,
        pltpu.SemaphoreType.DMA,
        pltpu.SemaphoreType.DMA,
        pltpu.SemaphoreType.DMA,
        pltpu.SemaphoreType.DMA,
    ],
)(_gather_body)


def _head_body(x_ref, w_ref, out_ref):
    x = x_ref[...][:, :D_MODEL]
    logits = lax.dot_general(
        x, w_ref[...],
        dimension_numbers=(((1,), (1,)), ((), ())),
        preferred_element_type=jnp.float32,
    )
    out_ref[...] = logits.reshape(BB, SEQ, VOCAB)


def _head_call(x):
    grid = BATCH // BB
    return pl.pallas_call(
        _head_body,
        grid=(grid,),
        in_specs=[
            pl.BlockSpec((ROWS_PER_BLK, D_PAD), lambda i: (i, 0)),
            pl.BlockSpec((VOCAB, D_MODEL), lambda i: (0, 0)),
        ],
        out_specs=pl.BlockSpec((BB, SEQ, VOCAB), lambda i: (i, 0, 0)),
        out_shape=jax.ShapeDtypeStruct((BATCH, SEQ, VOCAB), jnp.float32),
    )


def kernel(input_ids, embed_table, head_w):
    e_pad = jnp.pad(embed_table, ((0, 0), (0, D_PAD - D_MODEL)))
    idx = input_ids.reshape(-1).astype(jnp.int32)
    x = _gather_call(e_pad, idx)
    return _head_call(x)(x, head_w)


# speedup vs baseline: 5.1454x; 2.9636x over previous
import functools

import jax
import jax.numpy as jnp
from jax import lax
from jax.experimental import pallas as pl
from jax.experimental.pallas import tpu as pltpu
from jax.experimental.pallas import tpu_sc as plsc

VOCAB = 1000
D_MODEL = 64
D_PAD = 128
BATCH = 1024
SEQ = 50

B_TOTAL = BATCH * SEQ
NC, NS = 2, 16
NW = NC * NS
B_PER_W = B_TOTAL // NW
NBUF = 4
CHUNK = 80
NROUNDS = B_PER_W // (NBUF * CHUNK)


def _gather_body(e_hbm, idx_hbm, x_hbm, idx_v, rows_v,
                 g0, g1, g2, g3, w0, w1, w2, w3):
    gsems = [g0, g1, g2, g3]
    wsems = [w0, w1, w2, w3]
    wid = lax.axis_index("s") * NC + lax.axis_index("c")
    base = wid * B_PER_W
    pltpu.sync_copy(idx_hbm.at[pl.ds(base, B_PER_W)], idx_v)

    def fire_gather(chunk, s):
        pltpu.async_copy(
            e_hbm.at[idx_v.at[pl.ds(chunk * CHUNK, CHUNK)]],
            rows_v.at[s],
            gsems[s],
        )

    def wait_gather(s):
        pltpu.make_async_copy(
            e_hbm.at[pl.ds(0, CHUNK)], rows_v.at[s], gsems[s]
        ).wait()

    for s in range(NBUF):
        fire_gather(s, s)

    def round_body(j, _):
        first = j * NBUF
        writes = []
        for s in range(NBUF):
            wait_gather(s)
            writes.append(
                pltpu.async_copy(
                    rows_v.at[s],
                    x_hbm.at[pl.ds(base + (first + s) * CHUNK, CHUNK)],
                    wsems[s],
                )
            )
        for s in range(NBUF):
            writes[s].wait()

            @pl.when(j < NROUNDS - 1)
            def _():
                fire_gather(first + NBUF + s, s)

        return 0

    lax.fori_loop(0, NROUNDS, round_body, 0)


_gather_call = functools.partial(
    pl.kernel,
    out_type=jax.ShapeDtypeStruct((B_TOTAL, D_PAD), jnp.float32),
    mesh=plsc.VectorSubcoreMesh(core_axis_name="c", subcore_axis_name="s"),
    scratch_types=[
        pltpu.VMEM((B_PER_W,), jnp.int32),
        pltpu.VMEM((NBUF, CHUNK, D_PAD), jnp.float32),
        pltpu.SemaphoreType.DMA,
        pltpu.SemaphoreType.DMA,
        pltpu.SemaphoreType.DMA,
        pltpu.SemaphoreType.DMA,
        pltpu.SemaphoreType.DMA,
        pltpu.SemaphoreType.DMA,
        pltpu.SemaphoreType.DMA,
        pltpu.SemaphoreType.DMA,
    ],
)(_gather_body)


def _head_body(x_ref, w_ref, out_ref):
    xs = x_ref[0][:, :D_MODEL]
    out_ref[0] = lax.dot_general(
        w_ref[...], xs,
        dimension_numbers=(((1,), (1,)), ((), ())),
        preferred_element_type=jnp.float32,
    )


_head_call = pl.pallas_call(
    _head_body,
    grid=(SEQ,),
    in_specs=[
        pl.BlockSpec((1, BATCH, D_PAD), lambda i: (i, 0, 0)),
        pl.BlockSpec((VOCAB, D_MODEL), lambda i: (0, 0)),
    ],
    out_specs=pl.BlockSpec((1, VOCAB, BATCH), lambda i: (i, 0, 0)),
    out_shape=jax.ShapeDtypeStruct((SEQ, VOCAB, BATCH), jnp.float32),
)


def kernel(input_ids, embed_table, head_w):
    e_pad = jnp.pad(embed_table, ((0, 0), (0, D_PAD - D_MODEL)))
    idx = input_ids.T.reshape(-1).astype(jnp.int32)
    x = _gather_call(e_pad, idx)
    x3 = x.reshape(SEQ, BATCH, D_PAD)
    out_t = _head_call(x3, head_w)
    return jnp.transpose(out_t, (2, 0, 1))


# speedup vs baseline: 5.1819x; 1.0071x over previous
import functools

import jax
import jax.numpy as jnp
from jax import lax
from jax.experimental import pallas as pl
from jax.experimental.pallas import tpu as pltpu
from jax.experimental.pallas import tpu_sc as plsc

VOCAB = 1000
D_MODEL = 64
D_PAD = 128
BATCH = 1024
SEQ = 50

B_TOTAL = BATCH * SEQ
NC, NS = 2, 16
NW = NC * NS
NBUF = 4
NSPLIT = 2
SEQ_SPLIT = SEQ // NSPLIT
B_SPLIT = BATCH * SEQ_SPLIT
B_PER_W = B_SPLIT // NW
CHUNK = 40
NROUNDS = B_PER_W // (NBUF * CHUNK)


def _gather_body(e_hbm, idx_hbm, x_hbm, idx_v, rows_v,
                 g0, g1, g2, g3, w0, w1, w2, w3):
    gsems = [g0, g1, g2, g3]
    wsems = [w0, w1, w2, w3]
    wid = lax.axis_index("s") * NC + lax.axis_index("c")
    base = wid * B_PER_W
    pltpu.sync_copy(idx_hbm.at[pl.ds(base, B_PER_W)], idx_v)

    def fire_gather(chunk, s):
        pltpu.async_copy(
            e_hbm.at[idx_v.at[pl.ds(chunk * CHUNK, CHUNK)]],
            rows_v.at[s],
            gsems[s],
        )

    def wait_gather(s):
        pltpu.make_async_copy(
            e_hbm.at[pl.ds(0, CHUNK)], rows_v.at[s], gsems[s]
        ).wait()

    for s in range(NBUF):
        fire_gather(s, s)

    def round_body(j, _):
        first = j * NBUF
        writes = []
        for s in range(NBUF):
            wait_gather(s)
            writes.append(
                pltpu.async_copy(
                    rows_v.at[s],
                    x_hbm.at[pl.ds(base + (first + s) * CHUNK, CHUNK)],
                    wsems[s],
                )
            )
        for s in range(NBUF):
            writes[s].wait()

            @pl.when(j < NROUNDS - 1)
            def _():
                fire_gather(first + NBUF + s, s)

        return 0

    lax.fori_loop(0, NROUNDS, round_body, 0)


_gather_call = functools.partial(
    pl.kernel,
    out_type=jax.ShapeDtypeStruct((B_SPLIT, D_PAD), jnp.float32),
    mesh=plsc.VectorSubcoreMesh(core_axis_name="c", subcore_axis_name="s"),
    scratch_types=[
        pltpu.VMEM((B_PER_W,), jnp.int32),
        pltpu.VMEM((NBUF, CHUNK, D_PAD), jnp.float32),
        pltpu.SemaphoreType.DMA,
        pltpu.SemaphoreType.DMA,
        pltpu.SemaphoreType.DMA,
        pltpu.SemaphoreType.DMA,
        pltpu.SemaphoreType.DMA,
        pltpu.SemaphoreType.DMA,
        pltpu.SemaphoreType.DMA,
        pltpu.SemaphoreType.DMA,
    ],
)(_gather_body)


def _head_body(x_ref, w_ref, out_ref):
    xs = x_ref[0][:, :D_MODEL]
    out_ref[0] = lax.dot_general(
        w_ref[...], xs,
        dimension_numbers=(((1,), (1,)), ((), ())),
        preferred_element_type=jnp.float32,
    )


def _head_first(x3, w):
    return pl.pallas_call(
        _head_body,
        grid=(SEQ_SPLIT,),
        in_specs=[
            pl.BlockSpec((1, BATCH, D_PAD), lambda i: (i, 0, 0)),
            pl.BlockSpec((VOCAB, D_MODEL), lambda i: (0, 0)),
        ],
        out_specs=pl.BlockSpec((1, VOCAB, BATCH), lambda i: (i, 0, 0)),
        out_shape=jax.ShapeDtypeStruct((SEQ, VOCAB, BATCH), jnp.float32),
    )(x3, w)


def _head_second_body(prev_ref, x_ref, w_ref, out_ref):
    del prev_ref
    _head_body(x_ref, w_ref, out_ref)


def _head_second(prev, x3, w):
    return pl.pallas_call(
        _head_second_body,
        grid=(SEQ_SPLIT,),
        in_specs=[
            pl.BlockSpec(memory_space=pl.ANY),
            pl.BlockSpec((1, BATCH, D_PAD), lambda i: (i, 0, 0)),
            pl.BlockSpec((VOCAB, D_MODEL), lambda i: (0, 0)),
        ],
        out_specs=pl.BlockSpec((1, VOCAB, BATCH),
                               lambda i: (i + SEQ_SPLIT, 0, 0)),
        out_shape=jax.ShapeDtypeStruct((SEQ, VOCAB, BATCH), jnp.float32),
        input_output_aliases={0: 0},
    )(prev, x3, w)


def kernel(input_ids, embed_table, head_w):
    e_pad = jnp.pad(embed_table, ((0, 0), (0, D_PAD - D_MODEL)))
    idx = input_ids.T.reshape(-1).astype(jnp.int32)
    xs = [
        _gather_call(e_pad, lax.slice_in_dim(idx, k * B_SPLIT, (k + 1) * B_SPLIT))
        .reshape(SEQ_SPLIT, BATCH, D_PAD)
        for k in range(NSPLIT)
    ]
    out_t = _head_first(xs[0], head_w)
    out_t = _head_second(out_t, xs[1], head_w)
    return jnp.transpose(out_t, (2, 0, 1))


# speedup vs baseline: 5.2660x; 1.0162x over previous
import functools

import jax
import jax.numpy as jnp
from jax import lax
from jax.experimental import pallas as pl
from jax.experimental.pallas import tpu as pltpu
from jax.experimental.pallas import tpu_sc as plsc

VOCAB = 1000
D_MODEL = 64
D_PAD = 128
BATCH = 1024
SEQ = 50

B_TOTAL = BATCH * SEQ
NC, NS = 2, 16
NW = NC * NS
NBUF = 4
CHUNK = 40
SEQ0, SEQ1 = 15, 35


def _make_gather(n_rows):
    b_per_w = n_rows // NW
    nrounds = b_per_w // (NBUF * CHUNK)
    assert b_per_w == nrounds * NBUF * CHUNK

    def _gather_body(e_hbm, idx_hbm, x_hbm, idx_v, rows_v,
                     g0, g1, g2, g3, w0, w1, w2, w3):
        gsems = [g0, g1, g2, g3]
        wsems = [w0, w1, w2, w3]
        wid = lax.axis_index("s") * NC + lax.axis_index("c")
        base = wid * b_per_w
        pltpu.sync_copy(idx_hbm.at[pl.ds(base, b_per_w)], idx_v)

        def fire_gather(chunk, s):
            pltpu.async_copy(
                e_hbm.at[idx_v.at[pl.ds(chunk * CHUNK, CHUNK)]],
                rows_v.at[s],
                gsems[s],
            )

        def wait_gather(s):
            pltpu.make_async_copy(
                e_hbm.at[pl.ds(0, CHUNK)], rows_v.at[s], gsems[s]
            ).wait()

        for s in range(NBUF):
            fire_gather(s, s)

        def round_body(j, _):
            first = j * NBUF
            writes = []
            for s in range(NBUF):
                wait_gather(s)
                writes.append(
                    pltpu.async_copy(
                        rows_v.at[s],
                        x_hbm.at[pl.ds(base + (first + s) * CHUNK, CHUNK)],
                        wsems[s],
                    )
                )
            for s in range(NBUF):
                writes[s].wait()

                @pl.when(j < nrounds - 1)
                def _():
                    fire_gather(first + NBUF + s, s)

            return 0

        lax.fori_loop(0, nrounds, round_body, 0)

    return pl.kernel(
        _gather_body,
        out_type=jax.ShapeDtypeStruct((n_rows, D_PAD), jnp.float32),
        mesh=plsc.VectorSubcoreMesh(core_axis_name="c", subcore_axis_name="s"),
        scratch_types=[
            pltpu.VMEM((b_per_w,), jnp.int32),
            pltpu.VMEM((NBUF, CHUNK, D_PAD), jnp.float32),
            pltpu.SemaphoreType.DMA,
            pltpu.SemaphoreType.DMA,
            pltpu.SemaphoreType.DMA,
            pltpu.SemaphoreType.DMA,
            pltpu.SemaphoreType.DMA,
            pltpu.SemaphoreType.DMA,
            pltpu.SemaphoreType.DMA,
            pltpu.SemaphoreType.DMA,
        ],
    )


_gather_calls = {n: _make_gather(n * BATCH) for n in (SEQ0, SEQ1)}


def _head_body(x_ref, w_ref, out_ref):
    xs = x_ref[0][:, :D_MODEL]
    out_ref[0] = lax.dot_general(
        w_ref[...], xs,
        dimension_numbers=(((1,), (1,)), ((), ())),
        preferred_element_type=jnp.float32,
    )


def _head_first(x3, w):
    return pl.pallas_call(
        _head_body,
        grid=(SEQ0,),
        in_specs=[
            pl.BlockSpec((1, BATCH, D_PAD), lambda i: (i, 0, 0)),
            pl.BlockSpec((VOCAB, D_MODEL), lambda i: (0, 0)),
        ],
        out_specs=pl.BlockSpec((1, VOCAB, BATCH), lambda i: (i, 0, 0)),
        out_shape=jax.ShapeDtypeStruct((SEQ, VOCAB, BATCH), jnp.float32),
    )(x3, w)


def _head_second_body(prev_ref, x_ref, w_ref, out_ref):
    del prev_ref
    _head_body(x_ref, w_ref, out_ref)


def _head_second(prev, x3, w):
    return pl.pallas_call(
        _head_second_body,
        grid=(SEQ1,),
        in_specs=[
            pl.BlockSpec(memory_space=pl.ANY),
            pl.BlockSpec((1, BATCH, D_PAD), lambda i: (i, 0, 0)),
            pl.BlockSpec((VOCAB, D_MODEL), lambda i: (0, 0)),
        ],
        out_specs=pl.BlockSpec((1, VOCAB, BATCH),
                               lambda i: (i + SEQ0, 0, 0)),
        out_shape=jax.ShapeDtypeStruct((SEQ, VOCAB, BATCH), jnp.float32),
        input_output_aliases={0: 0},
    )(prev, x3, w)


def kernel(input_ids, embed_table, head_w):
    e_pad = jnp.pad(embed_table, ((0, 0), (0, D_PAD - D_MODEL)))
    idx = input_ids.T.reshape(-1).astype(jnp.int32)
    x0 = _gather_calls[SEQ0](e_pad, lax.slice_in_dim(idx, 0, SEQ0 * BATCH))
    x1 = _gather_calls[SEQ1](e_pad, lax.slice_in_dim(idx, SEQ0 * BATCH, SEQ * BATCH))
    out_t = _head_first(x0.reshape(SEQ0, BATCH, D_PAD), head_w)
    out_t = _head_second(out_t, x1.reshape(SEQ1, BATCH, D_PAD), head_w)
    return jnp.transpose(out_t, (2, 0, 1))
